# baseline (speedup 1.0000x reference)
import jax
import jax.numpy as jnp
from jax.experimental import pallas as pl
from jax.experimental.pallas import tpu as pltpu


def _mm_kernel(x_ref, cb_ref, o_ref):
    xb = x_ref[...].astype(jnp.bfloat16)
    cb = cb_ref[...].astype(jnp.bfloat16)
    o_ref[...] = jnp.dot(xb, cb, preferred_element_type=jnp.float32)


def kernel(x, codebook):
    B, N, K = x.shape
    D = codebook.shape[1]
    M = B * N
    bm = 512
    out = pl.pallas_call(
        _mm_kernel,
        grid=(M // bm,),
        in_specs=[
            pl.BlockSpec((bm, K), lambda i: (i, 0),
                         pipeline_mode=pl.Buffered(buffer_count=2)),
            pl.BlockSpec((K, D), lambda i: (0, 0)),
        ],
        out_specs=pl.BlockSpec((bm, D), lambda i: (i, 0)),
        out_shape=jax.ShapeDtypeStruct((M, D), jnp.float32),
        compiler_params=pltpu.CompilerParams(
            dimension_semantics=("arbitrary",),
        ),
    )(x.reshape(M, K), codebook)
    return out.reshape(B, N, D)
